# Initial kernel scaffold; baseline (speedup 1.0000x reference)
import jax
import jax.numpy as jnp
from jax.experimental import pallas as pl


def kernel(x, gate_W, gate_b, W1, b1, W2, b2):
    raise NotImplementedError("write your pallas kernel here")


# speedup vs baseline: 3.2526x; 3.2526x over previous
import functools

import jax
import jax.numpy as jnp
from jax import lax
from jax.experimental import pallas as pl
from jax.experimental.pallas import tpu as pltpu
from jax.experimental.pallas import tpu_sc as plsc

_B, _T, _C = 4, 2048, 1024
_E = 8
_H = 4 * _C
_N = _B * _T
_TB = 1024
_BLK = 256
_NBLK = _N // _BLK + _E
_P = _NBLK * _BLK
_NW = 32
_CHUNK = _N // _NW
_SUB = 64
_NSUB = _CHUNK // _SUB

_DN = (((1,), (0,)), ((), ()))


def _gate_body(x_ref, gw_ref, gb_ref, idx_ref, rank_ref, score_ref, cnt_ref,
               aux_ref, cnt_scr, imp_scr):
    b = pl.program_id(0)

    @pl.when(b == 0)
    def _init():
        cnt_scr[...] = jnp.zeros_like(cnt_scr)
        imp_scr[...] = jnp.zeros_like(imp_scr)

    logits = lax.dot_general(x_ref[...].astype(jnp.bfloat16),
                             gw_ref[...].astype(jnp.bfloat16), _DN,
                             preferred_element_type=jnp.float32)
    logits = logits + gb_ref[...]
    m = jnp.max(logits, axis=1, keepdims=True)
    unnorm = jnp.exp(logits - m)
    s = unnorm / jnp.sum(unnorm, axis=1, keepdims=True)
    sm = jnp.max(s, axis=1, keepdims=True)
    lane = lax.broadcasted_iota(jnp.int32, (_TB, _E), 1)
    idx = jnp.min(jnp.where(s >= sm, lane, _E), axis=1, keepdims=True)
    onehot = (lane == idx).astype(jnp.float32)
    r = lax.broadcasted_iota(jnp.int32, (_TB, _TB), 0)
    c = lax.broadcasted_iota(jnp.int32, (_TB, _TB), 1)
    tri = (r > c).astype(jnp.float32)
    before = lax.dot_general(tri, onehot, _DN,
                             preferred_element_type=jnp.float32)
    rank_in_blk = jnp.sum(before * onehot, axis=1, keepdims=True)
    run = cnt_scr[...]
    prev = jnp.sum(run * onehot, axis=1, keepdims=True)
    idx_ref[...] = idx
    rank_ref[...] = (rank_in_blk + prev).astype(jnp.int32)
    score_ref[...] = sm
    cnt_scr[...] = run + jnp.sum(onehot, axis=0, keepdims=True)
    imp_scr[...] = imp_scr[...] + jnp.sum(s, axis=0, keepdims=True)

    @pl.when(b == pl.num_programs(0) - 1)
    def _fin():
        cntf = cnt_scr[...]
        cnt_ref[...] = cntf.astype(jnp.int32)
        aux = _E * jnp.sum((imp_scr[...] / _N) * (cntf / _N))
        aux_ref[...] = aux.reshape(1, 1)


_gate = pl.pallas_call(
    _gate_body,
    grid=(_N // _TB,),
    in_specs=[
        pl.BlockSpec((_TB, _C), lambda b: (b, 0)),
        pl.BlockSpec((_C, _E), lambda b: (0, 0)),
        pl.BlockSpec((1, _E), lambda b: (0, 0)),
    ],
    out_specs=[
        pl.BlockSpec((_TB, 1), lambda b: (b, 0)),
        pl.BlockSpec((_TB, 1), lambda b: (b, 0)),
        pl.BlockSpec((_TB, 1), lambda b: (b, 0)),
        pl.BlockSpec((1, _E), lambda b: (0, 0)),
        pl.BlockSpec((1, 1), lambda b: (0, 0)),
    ],
    out_shape=[
        jax.ShapeDtypeStruct((_N, 1), jnp.int32),
        jax.ShapeDtypeStruct((_N, 1), jnp.int32),
        jax.ShapeDtypeStruct((_N, 1), jnp.float32),
        jax.ShapeDtypeStruct((1, _E), jnp.int32),
        jax.ShapeDtypeStruct((1, 1), jnp.float32),
    ],
    scratch_shapes=[
        pltpu.VMEM((1, _E), jnp.float32),
        pltpu.VMEM((1, _E), jnp.float32),
    ],
)

def _destmap_body(idx_ref, rank_ref, bases_ref, dest_ref):
    lane = lax.broadcasted_iota(jnp.int32, (_TB, _E), 1)
    onehot = (lane == idx_ref[...]).astype(jnp.int32)
    base = jnp.sum(onehot * bases_ref[...], axis=1, keepdims=True)
    dest_ref[...] = base + rank_ref[...]


_destmap = pl.pallas_call(
    _destmap_body,
    grid=(_N // _TB,),
    in_specs=[
        pl.BlockSpec((_TB, 1), lambda b: (b, 0)),
        pl.BlockSpec((_TB, 1), lambda b: (b, 0)),
        pl.BlockSpec((1, _E), lambda b: (0, 0)),
    ],
    out_specs=pl.BlockSpec((_TB, 1), lambda b: (b, 0)),
    out_shape=jax.ShapeDtypeStruct((_N, 1), jnp.int32),
)


@functools.cache
def _sc_kernels():
    mesh = plsc.VectorSubcoreMesh(core_axis_name="c", subcore_axis_name="s")

    @functools.partial(
        pl.kernel,
        mesh=mesh,
        out_type=jax.ShapeDtypeStruct((_P, _C), jnp.float32),
        scratch_types=[
            pltpu.VMEM((_NSUB, _SUB), jnp.int32),
            pltpu.VMEM((_SUB, _C), jnp.float32),
            pltpu.SemaphoreType.DMA,
        ],
    )
    def dispatch(x_hbm, dest_hbm, xs_hbm, dest_v, rows_v, sem):
        wid = lax.axis_index("s") * 2 + lax.axis_index("c")
        base = wid * _CHUNK
        pltpu.sync_copy(dest_hbm.at[wid], dest_v)
        for si in range(_NSUB):
            pltpu.sync_copy(x_hbm.at[pl.ds(base + si * _SUB, _SUB)], rows_v)
            pltpu.async_copy(rows_v, xs_hbm.at[dest_v.at[si]], sem).wait()

    @functools.partial(
        pl.kernel,
        mesh=mesh,
        out_type=jax.ShapeDtypeStruct((_N, _C), jnp.float32),
        scratch_types=[
            pltpu.VMEM((_NSUB, _SUB), jnp.int32),
            pltpu.VMEM((_SUB, _C), jnp.float32),
            pltpu.SemaphoreType.DMA,
        ],
    )
    def combine(hs_hbm, dest_hbm, out_hbm, dest_v, rows_v, sem):
        wid = lax.axis_index("s") * 2 + lax.axis_index("c")
        base = wid * _CHUNK
        pltpu.sync_copy(dest_hbm.at[wid], dest_v)
        for si in range(_NSUB):
            pltpu.async_copy(hs_hbm.at[dest_v.at[si]], rows_v, sem).wait()
            pltpu.sync_copy(rows_v, out_hbm.at[pl.ds(base + si * _SUB, _SUB)])

    return dispatch, combine


def _expert_body(eid_ref, act_ref, xmap_ref, xs_ref, w1_ref, b1_ref, w2_ref,
                 b2_ref, out_ref):
    b = pl.program_id(0)

    @pl.when(act_ref[b] == 1)
    def _():
        xb = xs_ref[...].astype(jnp.bfloat16)
        h = lax.dot_general(xb, w1_ref[0], _DN,
                            preferred_element_type=jnp.float32)
        h = h + b1_ref[0]
        h = jnp.maximum(h, 0.0).astype(jnp.bfloat16)
        o = lax.dot_general(h, w2_ref[0], _DN,
                            preferred_element_type=jnp.float32)
        out_ref[...] = o + b2_ref[0]


_expert = pl.pallas_call(
    _expert_body,
    grid_spec=pltpu.PrefetchScalarGridSpec(
        num_scalar_prefetch=3,
        grid=(_NBLK,),
        in_specs=[
            pl.BlockSpec((_BLK, _C), lambda b, eid, act, xm: (xm[b], 0)),
            pl.BlockSpec((1, _C, _H), lambda b, eid, act, xm: (eid[b], 0, 0)),
            pl.BlockSpec((1, 1, _H), lambda b, eid, act, xm: (eid[b], 0, 0)),
            pl.BlockSpec((1, _H, _C), lambda b, eid, act, xm: (eid[b], 0, 0)),
            pl.BlockSpec((1, 1, _C), lambda b, eid, act, xm: (eid[b], 0, 0)),
        ],
        out_specs=pl.BlockSpec((_BLK, _C), lambda b, eid, act, xm: (xm[b], 0)),
    ),
    out_shape=jax.ShapeDtypeStruct((_P, _C), jnp.float32),
)


def _finalize_body(h_ref, s_ref, o_ref):
    o_ref[...] = h_ref[...] * s_ref[...]


_finalize = pl.pallas_call(
    _finalize_body,
    grid=(_N // _TB,),
    in_specs=[
        pl.BlockSpec((_TB, _C), lambda b: (b, 0)),
        pl.BlockSpec((_TB, 1), lambda b: (b, 0)),
    ],
    out_specs=pl.BlockSpec((_TB, _C), lambda b: (b, 0)),
    out_shape=jax.ShapeDtypeStruct((_N, _C), jnp.float32),
)


def kernel(x, gate_W, gate_b, W1, b1, W2, b2):
    x2d = x.reshape(_N, _C)
    idx_col, rank_col, score_col, cnt, aux = _gate(x2d, gate_W,
                                                   gate_b.reshape(1, _E))
    counts = cnt.reshape(_E)
    nb = (counts + (_BLK - 1)) // _BLK
    cumnb = jnp.cumsum(nb)
    bases = jnp.concatenate(
        [jnp.zeros((1,), cumnb.dtype), cumnb[:-1]]).astype(jnp.int32) * _BLK
    total = cumnb[-1].astype(jnp.int32)
    bids = jnp.arange(_NBLK, dtype=jnp.int32)
    eid_raw = jnp.sum((bids[:, None] >= cumnb[None, :]).astype(jnp.int32),
                      axis=1)
    eid_last = jnp.sum((cumnb <= total - 1).astype(jnp.int32))
    active = (bids < total).astype(jnp.int32)
    eid = jnp.where(active == 1, eid_raw, eid_last).astype(jnp.int32)
    xmap = jnp.where(active == 1, bids, total - 1).astype(jnp.int32)

    dest3 = _destmap(idx_col, rank_col, bases.reshape(1, _E)).reshape(
        _NW, _NSUB, _SUB)
    _dispatch, _combine = _sc_kernels()
    xs = _dispatch(x2d, dest3)
    hs = _expert(eid, active, xmap, xs, W1.astype(jnp.bfloat16),
                 b1.reshape(_E, 1, _H), W2.astype(jnp.bfloat16),
                 b2.reshape(_E, 1, _C))
    hout = _combine(hs, dest3)
    out = _finalize(hout, score_col)
    return (out.reshape(_B, _T, _C), aux[0, 0])


# speedup vs baseline: 3.8728x; 1.1907x over previous
import functools

import jax
import jax.numpy as jnp
from jax import lax
from jax.experimental import pallas as pl
from jax.experimental.pallas import tpu as pltpu
from jax.experimental.pallas import tpu_sc as plsc

_B, _T, _C = 4, 2048, 1024
_E = 8
_H = 4 * _C
_N = _B * _T
_TB = 1024
_BLK = 256
_NBLK = _N // _BLK + _E
_P = _NBLK * _BLK
_NW = 32
_CHUNK = _N // _NW
_SUB = 64
_NSUB = _CHUNK // _SUB

_DN = (((1,), (0,)), ((), ()))


def _gate_body(x_ref, gw_ref, gb_ref, idx_ref, rank_ref, score_ref, cnt_ref,
               aux_ref, cnt_scr, imp_scr):
    b = pl.program_id(0)

    @pl.when(b == 0)
    def _init():
        cnt_scr[...] = jnp.zeros_like(cnt_scr)
        imp_scr[...] = jnp.zeros_like(imp_scr)

    logits = lax.dot_general(x_ref[...], gw_ref[...], _DN,
                             preferred_element_type=jnp.float32)
    logits = logits + gb_ref[...]
    m = jnp.max(logits, axis=1, keepdims=True)
    unnorm = jnp.exp(logits - m)
    s = unnorm / jnp.sum(unnorm, axis=1, keepdims=True)
    sm = jnp.max(s, axis=1, keepdims=True)
    lane = lax.broadcasted_iota(jnp.int32, (_TB, _E), 1)
    idx = jnp.min(jnp.where(s >= sm, lane, _E), axis=1, keepdims=True)
    onehot = (lane == idx).astype(jnp.float32)
    r = lax.broadcasted_iota(jnp.int32, (_TB, _TB), 0)
    c = lax.broadcasted_iota(jnp.int32, (_TB, _TB), 1)
    tri = (r > c).astype(jnp.float32)
    before = lax.dot_general(tri, onehot, _DN,
                             preferred_element_type=jnp.float32)
    rank_in_blk = jnp.sum(before * onehot, axis=1, keepdims=True)
    run = cnt_scr[...]
    prev = jnp.sum(run * onehot, axis=1, keepdims=True)
    idx_ref[...] = idx
    rank_ref[...] = (rank_in_blk + prev).astype(jnp.int32)
    score_ref[...] = jnp.broadcast_to(sm, (_TB, 128))
    cnt_scr[...] = run + jnp.sum(onehot, axis=0, keepdims=True)
    imp_scr[...] = imp_scr[...] + jnp.sum(s, axis=0, keepdims=True)

    @pl.when(b == pl.num_programs(0) - 1)
    def _fin():
        cntf = cnt_scr[...]
        cnt_ref[...] = cntf.astype(jnp.int32)
        aux = _E * jnp.sum((imp_scr[...] / _N) * (cntf / _N))
        aux_ref[...] = aux.reshape(1, 1)


_gate = pl.pallas_call(
    _gate_body,
    grid=(_N // _TB,),
    in_specs=[
        pl.BlockSpec((_TB, _C), lambda b: (b, 0)),
        pl.BlockSpec((_C, _E), lambda b: (0, 0)),
        pl.BlockSpec((1, _E), lambda b: (0, 0)),
    ],
    out_specs=[
        pl.BlockSpec((_TB, 1), lambda b: (b, 0)),
        pl.BlockSpec((_TB, 1), lambda b: (b, 0)),
        pl.BlockSpec((_TB, 128), lambda b: (b, 0)),
        pl.BlockSpec((1, _E), lambda b: (0, 0)),
        pl.BlockSpec((1, 1), lambda b: (0, 0)),
    ],
    out_shape=[
        jax.ShapeDtypeStruct((_N, 1), jnp.int32),
        jax.ShapeDtypeStruct((_N, 1), jnp.int32),
        jax.ShapeDtypeStruct((_N, 128), jnp.float32),
        jax.ShapeDtypeStruct((1, _E), jnp.int32),
        jax.ShapeDtypeStruct((1, 1), jnp.float32),
    ],
    scratch_shapes=[
        pltpu.VMEM((1, _E), jnp.float32),
        pltpu.VMEM((1, _E), jnp.float32),
    ],
)

def _destmap_body(idx_ref, rank_ref, bases_ref, dest_ref):
    lane = lax.broadcasted_iota(jnp.int32, (_TB, _E), 1)
    onehot = (lane == idx_ref[...]).astype(jnp.int32)
    base = jnp.sum(onehot * bases_ref[...], axis=1, keepdims=True)
    dest_ref[...] = base + rank_ref[...]


_destmap = pl.pallas_call(
    _destmap_body,
    grid=(_N // _TB,),
    in_specs=[
        pl.BlockSpec((_TB, 1), lambda b: (b, 0)),
        pl.BlockSpec((_TB, 1), lambda b: (b, 0)),
        pl.BlockSpec((1, _E), lambda b: (0, 0)),
    ],
    out_specs=pl.BlockSpec((_TB, 1), lambda b: (b, 0)),
    out_shape=jax.ShapeDtypeStruct((_N, 1), jnp.int32),
)


@functools.cache
def _sc_kernels():
    mesh = plsc.VectorSubcoreMesh(core_axis_name="c", subcore_axis_name="s")

    @functools.partial(
        pl.kernel,
        mesh=mesh,
        out_type=[
            jax.ShapeDtypeStruct((_P, _C), jnp.float32),
            jax.ShapeDtypeStruct((_P, 128), jnp.float32),
        ],
        scratch_types=[
            pltpu.VMEM((_NSUB, _SUB), jnp.int32),
            pltpu.VMEM((_SUB, _C), jnp.float32),
            pltpu.VMEM((_SUB, 128), jnp.float32),
            pltpu.SemaphoreType.DMA,
        ],
    )
    def dispatch(x_hbm, sc16_hbm, dest_hbm, xs_hbm, ss_hbm, dest_v, rows_v,
                 srow_v, sem):
        wid = lax.axis_index("s") * 2 + lax.axis_index("c")
        base = wid * _CHUNK
        pltpu.sync_copy(dest_hbm.at[wid], dest_v)
        for si in range(_NSUB):
            pltpu.sync_copy(x_hbm.at[pl.ds(base + si * _SUB, _SUB)], rows_v)
            pltpu.async_copy(rows_v, xs_hbm.at[dest_v.at[si]], sem).wait()
            pltpu.sync_copy(sc16_hbm.at[pl.ds(base + si * _SUB, _SUB)], srow_v)
            pltpu.async_copy(srow_v, ss_hbm.at[dest_v.at[si]], sem).wait()

    @functools.partial(
        pl.kernel,
        mesh=mesh,
        out_type=jax.ShapeDtypeStruct((_N, _C), jnp.float32),
        scratch_types=[
            pltpu.VMEM((_NSUB, _SUB), jnp.int32),
            pltpu.VMEM((_SUB, _C), jnp.float32),
            pltpu.SemaphoreType.DMA,
        ],
    )
    def combine(hs_hbm, dest_hbm, out_hbm, dest_v, rows_v, sem):
        wid = lax.axis_index("s") * 2 + lax.axis_index("c")
        base = wid * _CHUNK
        pltpu.sync_copy(dest_hbm.at[wid], dest_v)
        for si in range(_NSUB):
            pltpu.async_copy(hs_hbm.at[dest_v.at[si]], rows_v, sem).wait()
            pltpu.sync_copy(rows_v, out_hbm.at[pl.ds(base + si * _SUB, _SUB)])

    return dispatch, combine


_HH = _H // 2


def _expert_a_body(eid_ref, act_ref, xmap_ref, xs_ref, w1_ref, b1_ref, w2_ref,
                   out_ref):
    b = pl.program_id(0)

    @pl.when(act_ref[b] == 1)
    def _():
        h = lax.dot_general(xs_ref[...], w1_ref[0], _DN,
                            preferred_element_type=jnp.float32)
        h = jnp.maximum(h + b1_ref[0], 0.0)
        out_ref[...] = lax.dot_general(h, w2_ref[0], _DN,
                                       preferred_element_type=jnp.float32)


def _expert_b_body(eid_ref, act_ref, xmap_ref, xs_ref, ss_ref, prev_ref,
                   w1_ref, b1_ref, w2_ref, b2_ref, out_ref):
    b = pl.program_id(0)

    @pl.when(act_ref[b] == 1)
    def _():
        h = lax.dot_general(xs_ref[...], w1_ref[0], _DN,
                            preferred_element_type=jnp.float32)
        h = jnp.maximum(h + b1_ref[0], 0.0)
        o = lax.dot_general(h, w2_ref[0], _DN,
                            preferred_element_type=jnp.float32)
        out_ref[...] = (prev_ref[...] + o + b2_ref[0]) * ss_ref[:, 0:1]


_expert_a = pl.pallas_call(
    _expert_a_body,
    grid_spec=pltpu.PrefetchScalarGridSpec(
        num_scalar_prefetch=3,
        grid=(_NBLK,),
        in_specs=[
            pl.BlockSpec((_BLK, _C), lambda b, eid, act, xm: (xm[b], 0)),
            pl.BlockSpec((1, _C, _HH), lambda b, eid, act, xm: (eid[b], 0, 0)),
            pl.BlockSpec((1, 1, _HH), lambda b, eid, act, xm: (eid[b], 0, 0)),
            pl.BlockSpec((1, _HH, _C), lambda b, eid, act, xm: (eid[b], 0, 0)),
        ],
        out_specs=pl.BlockSpec((_BLK, _C), lambda b, eid, act, xm: (xm[b], 0)),
    ),
    out_shape=jax.ShapeDtypeStruct((_P, _C), jnp.float32),
)

_expert_b = pl.pallas_call(
    _expert_b_body,
    grid_spec=pltpu.PrefetchScalarGridSpec(
        num_scalar_prefetch=3,
        grid=(_NBLK,),
        in_specs=[
            pl.BlockSpec((_BLK, _C), lambda b, eid, act, xm: (xm[b], 0)),
            pl.BlockSpec((_BLK, 128), lambda b, eid, act, xm: (xm[b], 0)),
            pl.BlockSpec((_BLK, _C), lambda b, eid, act, xm: (xm[b], 0)),
            pl.BlockSpec((1, _C, _HH), lambda b, eid, act, xm: (eid[b], 0, 1)),
            pl.BlockSpec((1, 1, _HH), lambda b, eid, act, xm: (eid[b], 0, 1)),
            pl.BlockSpec((1, _HH, _C), lambda b, eid, act, xm: (eid[b], 1, 0)),
            pl.BlockSpec((1, 1, _C), lambda b, eid, act, xm: (eid[b], 0, 0)),
        ],
        out_specs=pl.BlockSpec((_BLK, _C), lambda b, eid, act, xm: (xm[b], 0)),
    ),
    out_shape=jax.ShapeDtypeStruct((_P, _C), jnp.float32),
)


def kernel(x, gate_W, gate_b, W1, b1, W2, b2):
    x2d = x.reshape(_N, _C)
    idx_col, rank_col, score_col, cnt, aux = _gate(x2d, gate_W,
                                                   gate_b.reshape(1, _E))
    counts = cnt.reshape(_E)
    nb = (counts + (_BLK - 1)) // _BLK
    cumnb = jnp.cumsum(nb)
    bases = jnp.concatenate(
        [jnp.zeros((1,), cumnb.dtype), cumnb[:-1]]).astype(jnp.int32) * _BLK
    total = cumnb[-1].astype(jnp.int32)
    bids = jnp.arange(_NBLK, dtype=jnp.int32)
    eid_raw = jnp.sum((bids[:, None] >= cumnb[None, :]).astype(jnp.int32),
                      axis=1)
    eid_last = jnp.sum((cumnb <= total - 1).astype(jnp.int32))
    active = (bids < total).astype(jnp.int32)
    eid = jnp.where(active == 1, eid_raw, eid_last).astype(jnp.int32)
    xmap = jnp.where(active == 1, bids, total - 1).astype(jnp.int32)

    dest3 = _destmap(idx_col, rank_col, bases.reshape(1, _E)).reshape(
        _NW, _NSUB, _SUB)
    _dispatch, _combine = _sc_kernels()
    xs, ss = _dispatch(x2d, score_col, dest3)
    b1r = b1.reshape(_E, 1, _H)
    hs_a = _expert_a(eid, active, xmap, xs, W1, b1r, W2)
    hs = _expert_b(eid, active, xmap, xs, ss, hs_a, W1, b1r, W2,
                   b2.reshape(_E, 1, _C))
    hout = _combine(hs, dest3)
    return (hout.reshape(_B, _T, _C), aux[0, 0])


# speedup vs baseline: 4.1058x; 1.0602x over previous
import functools

import jax
import jax.numpy as jnp
from jax import lax
from jax.experimental import pallas as pl
from jax.experimental.pallas import tpu as pltpu
from jax.experimental.pallas import tpu_sc as plsc

_B, _T, _C = 4, 2048, 1024
_E = 8
_H = 4 * _C
_N = _B * _T
_TB = 1024
_BLK = 512
_NBLK = _N // _BLK + _E
_P = _NBLK * _BLK
_NW = 32
_CHUNK = _N // _NW
_SUB = 32
_NSUB = _CHUNK // _SUB

_DN = (((1,), (0,)), ((), ()))


def _gate_body(x_ref, gw_ref, gb_ref, idx_ref, rank_ref, score_ref, cnt_ref,
               aux_ref, cnt_scr, imp_scr):
    b = pl.program_id(0)

    @pl.when(b == 0)
    def _init():
        cnt_scr[...] = jnp.zeros_like(cnt_scr)
        imp_scr[...] = jnp.zeros_like(imp_scr)

    logits = lax.dot_general(x_ref[...], gw_ref[...], _DN,
                             preferred_element_type=jnp.float32)
    logits = logits + gb_ref[...]
    m = jnp.max(logits, axis=1, keepdims=True)
    unnorm = jnp.exp(logits - m)
    s = unnorm / jnp.sum(unnorm, axis=1, keepdims=True)
    sm = jnp.max(s, axis=1, keepdims=True)
    lane = lax.broadcasted_iota(jnp.int32, (_TB, _E), 1)
    idx = jnp.min(jnp.where(s >= sm, lane, _E), axis=1, keepdims=True)
    onehot = (lane == idx).astype(jnp.float32)
    r = lax.broadcasted_iota(jnp.int32, (_TB, _TB), 0)
    c = lax.broadcasted_iota(jnp.int32, (_TB, _TB), 1)
    tri = (r > c).astype(jnp.float32)
    before = lax.dot_general(tri, onehot, _DN,
                             preferred_element_type=jnp.float32)
    rank_in_blk = jnp.sum(before * onehot, axis=1, keepdims=True)
    run = cnt_scr[...]
    prev = jnp.sum(run * onehot, axis=1, keepdims=True)
    idx_ref[...] = idx
    rank_ref[...] = (rank_in_blk + prev).astype(jnp.int32)
    score_ref[...] = jnp.broadcast_to(sm, (_TB, 128))
    cnt_scr[...] = run + jnp.sum(onehot, axis=0, keepdims=True)
    imp_scr[...] = imp_scr[...] + jnp.sum(s, axis=0, keepdims=True)

    @pl.when(b == pl.num_programs(0) - 1)
    def _fin():
        cntf = cnt_scr[...]
        cnt_ref[...] = cntf.astype(jnp.int32)
        aux = _E * jnp.sum((imp_scr[...] / _N) * (cntf / _N))
        aux_ref[...] = aux.reshape(1, 1)


_gate = pl.pallas_call(
    _gate_body,
    grid=(_N // _TB,),
    in_specs=[
        pl.BlockSpec((_TB, _C), lambda b: (b, 0)),
        pl.BlockSpec((_C, _E), lambda b: (0, 0)),
        pl.BlockSpec((1, _E), lambda b: (0, 0)),
    ],
    out_specs=[
        pl.BlockSpec((_TB, 1), lambda b: (b, 0)),
        pl.BlockSpec((_TB, 1), lambda b: (b, 0)),
        pl.BlockSpec((_TB, 128), lambda b: (b, 0)),
        pl.BlockSpec((1, _E), lambda b: (0, 0)),
        pl.BlockSpec((1, 1), lambda b: (0, 0)),
    ],
    out_shape=[
        jax.ShapeDtypeStruct((_N, 1), jnp.int32),
        jax.ShapeDtypeStruct((_N, 1), jnp.int32),
        jax.ShapeDtypeStruct((_N, 128), jnp.float32),
        jax.ShapeDtypeStruct((1, _E), jnp.int32),
        jax.ShapeDtypeStruct((1, 1), jnp.float32),
    ],
    scratch_shapes=[
        pltpu.VMEM((1, _E), jnp.float32),
        pltpu.VMEM((1, _E), jnp.float32),
    ],
)

def _destmap_body(idx_ref, rank_ref, bases_ref, dest_ref):
    lane = lax.broadcasted_iota(jnp.int32, (_TB, _E), 1)
    onehot = (lane == idx_ref[...]).astype(jnp.int32)
    base = jnp.sum(onehot * bases_ref[...], axis=1, keepdims=True)
    dest_ref[...] = base + rank_ref[...]


_destmap = pl.pallas_call(
    _destmap_body,
    grid=(_N // _TB,),
    in_specs=[
        pl.BlockSpec((_TB, 1), lambda b: (b, 0)),
        pl.BlockSpec((_TB, 1), lambda b: (b, 0)),
        pl.BlockSpec((1, _E), lambda b: (0, 0)),
    ],
    out_specs=pl.BlockSpec((_TB, 1), lambda b: (b, 0)),
    out_shape=jax.ShapeDtypeStruct((_N, 1), jnp.int32),
)


@functools.cache
def _sc_kernels():
    mesh = plsc.VectorSubcoreMesh(core_axis_name="c", subcore_axis_name="s")

    @functools.partial(
        pl.kernel,
        mesh=mesh,
        out_type=[
            jax.ShapeDtypeStruct((_P, _C), jnp.float32),
            jax.ShapeDtypeStruct((_P, 128), jnp.float32),
        ],
        scratch_types=[
            pltpu.VMEM((_NSUB, _SUB), jnp.int32),
            pltpu.VMEM((2, _SUB, _C), jnp.float32),
            pltpu.VMEM((2, _SUB, 128), jnp.float32),
            pltpu.SemaphoreType.DMA,
        ],
    )
    def dispatch(x_hbm, sc16_hbm, dest_hbm, xs_hbm, ss_hbm, dest_v, rows_v,
                 srow_v, sem):
        wid = lax.axis_index("s") * 2 + lax.axis_index("c")
        base = wid * _CHUNK
        pltpu.sync_copy(dest_hbm.at[wid], dest_v)
        pltpu.sync_copy(x_hbm.at[pl.ds(base, _SUB)], rows_v.at[0])
        pltpu.sync_copy(sc16_hbm.at[pl.ds(base, _SUB)], srow_v.at[0])
        for si in range(_NSUB):
            cur = si % 2
            h1 = pltpu.async_copy(rows_v.at[cur], xs_hbm.at[dest_v.at[si]],
                                  sem)
            h2 = pltpu.async_copy(srow_v.at[cur], ss_hbm.at[dest_v.at[si]],
                                  sem)
            if si + 1 < _NSUB:
                nxt = (si + 1) % 2
                off = base + (si + 1) * _SUB
                pltpu.sync_copy(x_hbm.at[pl.ds(off, _SUB)], rows_v.at[nxt])
                pltpu.sync_copy(sc16_hbm.at[pl.ds(off, _SUB)], srow_v.at[nxt])
            h1.wait()
            h2.wait()

    @functools.partial(
        pl.kernel,
        mesh=mesh,
        out_type=jax.ShapeDtypeStruct((_N, _C), jnp.float32),
        scratch_types=[
            pltpu.VMEM((_NSUB, _SUB), jnp.int32),
            pltpu.VMEM((2, _SUB, _C), jnp.float32),
            pltpu.SemaphoreType.DMA,
            pltpu.SemaphoreType.DMA,
        ],
    )
    def combine(hs_hbm, dest_hbm, out_hbm, dest_v, rows_v, sem0, sem1):
        wid = lax.axis_index("s") * 2 + lax.axis_index("c")
        base = wid * _CHUNK
        sems = (sem0, sem1)
        pltpu.sync_copy(dest_hbm.at[wid], dest_v)
        pending = pltpu.async_copy(hs_hbm.at[dest_v.at[0]], rows_v.at[0],
                                   sems[0])
        for si in range(_NSUB):
            cur = si % 2
            if si + 1 < _NSUB:
                nxt_h = pltpu.async_copy(hs_hbm.at[dest_v.at[si + 1]],
                                         rows_v.at[(si + 1) % 2],
                                         sems[(si + 1) % 2])
            pending.wait()
            pltpu.sync_copy(rows_v.at[cur],
                            out_hbm.at[pl.ds(base + si * _SUB, _SUB)])
            if si + 1 < _NSUB:
                pending = nxt_h

    return dispatch, combine


_HH = _H // 2


def _expert_a_body(eid_ref, act_ref, xmap_ref, xs_ref, w1_ref, b1_ref, w2_ref,
                   out_ref):
    b = pl.program_id(0)

    @pl.when(act_ref[b] == 1)
    def _():
        h = lax.dot_general(xs_ref[...], w1_ref[0], _DN,
                            preferred_element_type=jnp.float32)
        h = jnp.maximum(h + b1_ref[0], 0.0)
        out_ref[...] = lax.dot_general(h, w2_ref[0], _DN,
                                       preferred_element_type=jnp.float32)


def _expert_b_body(eid_ref, act_ref, xmap_ref, xs_ref, ss_ref, prev_ref,
                   w1_ref, b1_ref, w2_ref, b2_ref, out_ref):
    b = pl.program_id(0)

    @pl.when(act_ref[b] == 1)
    def _():
        h = lax.dot_general(xs_ref[...], w1_ref[0], _DN,
                            preferred_element_type=jnp.float32)
        h = jnp.maximum(h + b1_ref[0], 0.0)
        o = lax.dot_general(h, w2_ref[0], _DN,
                            preferred_element_type=jnp.float32)
        out_ref[...] = (prev_ref[...] + o + b2_ref[0]) * ss_ref[:, 0:1]


_expert_a = pl.pallas_call(
    _expert_a_body,
    grid_spec=pltpu.PrefetchScalarGridSpec(
        num_scalar_prefetch=3,
        grid=(_NBLK,),
        in_specs=[
            pl.BlockSpec((_BLK, _C), lambda b, eid, act, xm: (xm[b], 0)),
            pl.BlockSpec((1, _C, _HH), lambda b, eid, act, xm: (eid[b], 0, 0)),
            pl.BlockSpec((1, 1, _HH), lambda b, eid, act, xm: (eid[b], 0, 0)),
            pl.BlockSpec((1, _HH, _C), lambda b, eid, act, xm: (eid[b], 0, 0)),
        ],
        out_specs=pl.BlockSpec((_BLK, _C), lambda b, eid, act, xm: (xm[b], 0)),
    ),
    out_shape=jax.ShapeDtypeStruct((_P, _C), jnp.float32),
)

_expert_b = pl.pallas_call(
    _expert_b_body,
    grid_spec=pltpu.PrefetchScalarGridSpec(
        num_scalar_prefetch=3,
        grid=(_NBLK,),
        in_specs=[
            pl.BlockSpec((_BLK, _C), lambda b, eid, act, xm: (xm[b], 0)),
            pl.BlockSpec((_BLK, 128), lambda b, eid, act, xm: (xm[b], 0)),
            pl.BlockSpec((_BLK, _C), lambda b, eid, act, xm: (xm[b], 0)),
            pl.BlockSpec((1, _C, _HH), lambda b, eid, act, xm: (eid[b], 0, 1)),
            pl.BlockSpec((1, 1, _HH), lambda b, eid, act, xm: (eid[b], 0, 1)),
            pl.BlockSpec((1, _HH, _C), lambda b, eid, act, xm: (eid[b], 1, 0)),
            pl.BlockSpec((1, 1, _C), lambda b, eid, act, xm: (eid[b], 0, 0)),
        ],
        out_specs=pl.BlockSpec((_BLK, _C), lambda b, eid, act, xm: (xm[b], 0)),
    ),
    out_shape=jax.ShapeDtypeStruct((_P, _C), jnp.float32),
)


def kernel(x, gate_W, gate_b, W1, b1, W2, b2):
    x2d = x.reshape(_N, _C)
    idx_col, rank_col, score_col, cnt, aux = _gate(x2d, gate_W,
                                                   gate_b.reshape(1, _E))
    counts = cnt.reshape(_E)
    nb = (counts + (_BLK - 1)) // _BLK
    cumnb = jnp.cumsum(nb)
    bases = jnp.concatenate(
        [jnp.zeros((1,), cumnb.dtype), cumnb[:-1]]).astype(jnp.int32) * _BLK
    total = cumnb[-1].astype(jnp.int32)
    bids = jnp.arange(_NBLK, dtype=jnp.int32)
    eid_raw = jnp.sum((bids[:, None] >= cumnb[None, :]).astype(jnp.int32),
                      axis=1)
    eid_last = jnp.sum((cumnb <= total - 1).astype(jnp.int32))
    active = (bids < total).astype(jnp.int32)
    eid = jnp.where(active == 1, eid_raw, eid_last).astype(jnp.int32)
    xmap = jnp.where(active == 1, bids, total - 1).astype(jnp.int32)

    dest3 = _destmap(idx_col, rank_col, bases.reshape(1, _E)).reshape(
        _NW, _NSUB, _SUB)
    _dispatch, _combine = _sc_kernels()
    xs, ss = _dispatch(x2d, score_col, dest3)
    b1r = b1.reshape(_E, 1, _H)
    hs_a = _expert_a(eid, active, xmap, xs, W1, b1r, W2)
    hs = _expert_b(eid, active, xmap, xs, ss, hs_a, W1, b1r, W2,
                   b2.reshape(_E, 1, _C))
    hout = _combine(hs, dest3)
    return (hout.reshape(_B, _T, _C), aux[0, 0])


# speedup vs baseline: 4.1902x; 1.0206x over previous
import functools

import jax
import jax.numpy as jnp
from jax import lax
from jax.experimental import pallas as pl
from jax.experimental.pallas import tpu as pltpu
from jax.experimental.pallas import tpu_sc as plsc

_B, _T, _C = 4, 2048, 1024
_E = 8
_H = 4 * _C
_N = _B * _T
_TB = 1024
_BLK = 512
_NBLK = _N // _BLK + _E
_P = _NBLK * _BLK
_NW = 32
_CHUNK = _N // _NW
_SUB = 32
_NSUB = _CHUNK // _SUB

_DN = (((1,), (0,)), ((), ()))


def _gate_body(x_ref, gw_ref, gb_ref, score_ref, dest_ref, cnt_ref, aux_ref,
               cnt_scr, imp_scr, idx_scr, rank_scr):
    p = pl.program_id(0)
    b = pl.program_id(1)

    @pl.when((p == 0) & (b == 0))
    def _init():
        cnt_scr[...] = jnp.zeros_like(cnt_scr)
        imp_scr[...] = jnp.zeros_like(imp_scr)

    @pl.when(p == 0)
    def _phase0():
        logits = lax.dot_general(x_ref[...], gw_ref[...], _DN,
                                 preferred_element_type=jnp.float32)
        logits = logits + gb_ref[...]
        m = jnp.max(logits, axis=1, keepdims=True)
        unnorm = jnp.exp(logits - m)
        s = unnorm / jnp.sum(unnorm, axis=1, keepdims=True)
        sm = jnp.max(s, axis=1, keepdims=True)
        lane = lax.broadcasted_iota(jnp.int32, (_TB, _E), 1)
        idx = jnp.min(jnp.where(s >= sm, lane, _E), axis=1, keepdims=True)
        onehot = (lane == idx).astype(jnp.float32)
        r = lax.broadcasted_iota(jnp.int32, (_TB, _TB), 0)
        c = lax.broadcasted_iota(jnp.int32, (_TB, _TB), 1)
        tri = (r > c).astype(jnp.float32)
        before = lax.dot_general(tri, onehot, _DN,
                                 preferred_element_type=jnp.float32)
        rank_in_blk = jnp.sum(before * onehot, axis=1, keepdims=True)
        run = cnt_scr[...]
        prev = jnp.sum(run * onehot, axis=1, keepdims=True)
        idx_scr[pl.ds(b * _TB, _TB), :] = idx
        rank_scr[pl.ds(b * _TB, _TB), :] = (rank_in_blk + prev).astype(
            jnp.int32)
        score_ref[...] = jnp.broadcast_to(sm, (_TB, 128))
        cnt_scr[...] = run + jnp.sum(onehot, axis=0, keepdims=True)
        imp_scr[...] = imp_scr[...] + jnp.sum(s, axis=0, keepdims=True)

    @pl.when(p == 1)
    def _phase1():
        cntf = cnt_scr[...]
        nb = jnp.floor((cntf + (_BLK - 1)) * (1.0 / _BLK))
        er = lax.broadcasted_iota(jnp.int32, (_E, _E), 0)
        ec = lax.broadcasted_iota(jnp.int32, (_E, _E), 1)
        tri8 = (er <= ec).astype(jnp.float32)
        cum = lax.dot_general(nb, tri8, _DN,
                              preferred_element_type=jnp.float32)
        bases = (cum - nb) * _BLK
        idx_b = idx_scr[pl.ds(b * _TB, _TB), :]
        rank_b = rank_scr[pl.ds(b * _TB, _TB), :]
        lane = lax.broadcasted_iota(jnp.int32, (_TB, _E), 1)
        onehot = (lane == idx_b).astype(jnp.float32)
        base_tok = jnp.sum(onehot * bases, axis=1, keepdims=True)
        dest_ref[...] = base_tok.astype(jnp.int32) + rank_b

    @pl.when((p == 1) & (b == pl.num_programs(1) - 1))
    def _fin():
        cntf = cnt_scr[...]
        cnt_ref[...] = cntf.astype(jnp.int32)
        aux = _E * jnp.sum((imp_scr[...] / _N) * (cntf / _N))
        aux_ref[...] = aux.reshape(1, 1)


_NGB = _N // _TB

_gate = pl.pallas_call(
    _gate_body,
    grid=(2, _NGB),
    in_specs=[
        pl.BlockSpec((_TB, _C), lambda p, b: (b * (1 - p) + (_NGB - 1) * p, 0)),
        pl.BlockSpec((_C, _E), lambda p, b: (0, 0)),
        pl.BlockSpec((1, _E), lambda p, b: (0, 0)),
    ],
    out_specs=[
        pl.BlockSpec((_TB, 128), lambda p, b: (b * (1 - p) + (_NGB - 1) * p, 0)),
        pl.BlockSpec((_TB, 1), lambda p, b: (b * p, 0)),
        pl.BlockSpec((1, _E), lambda p, b: (0, 0)),
        pl.BlockSpec((1, 1), lambda p, b: (0, 0)),
    ],
    out_shape=[
        jax.ShapeDtypeStruct((_N, 128), jnp.float32),
        jax.ShapeDtypeStruct((_N, 1), jnp.int32),
        jax.ShapeDtypeStruct((1, _E), jnp.int32),
        jax.ShapeDtypeStruct((1, 1), jnp.float32),
    ],
    scratch_shapes=[
        pltpu.VMEM((1, _E), jnp.float32),
        pltpu.VMEM((1, _E), jnp.float32),
        pltpu.VMEM((_N, 1), jnp.int32),
        pltpu.VMEM((_N, 1), jnp.int32),
    ],
)

@functools.cache
def _sc_kernels():
    mesh = plsc.VectorSubcoreMesh(core_axis_name="c", subcore_axis_name="s")

    @functools.partial(
        pl.kernel,
        mesh=mesh,
        out_type=[
            jax.ShapeDtypeStruct((_P, _C), jnp.float32),
            jax.ShapeDtypeStruct((_P, 128), jnp.float32),
        ],
        scratch_types=[
            pltpu.VMEM((_NSUB, _SUB), jnp.int32),
            pltpu.VMEM((2, _SUB, _C), jnp.float32),
            pltpu.VMEM((2, _SUB, 128), jnp.float32),
            pltpu.SemaphoreType.DMA,
        ],
    )
    def dispatch(x_hbm, sc16_hbm, dest_hbm, xs_hbm, ss_hbm, dest_v, rows_v,
                 srow_v, sem):
        wid = lax.axis_index("s") * 2 + lax.axis_index("c")
        base = wid * _CHUNK
        pltpu.sync_copy(dest_hbm.at[wid], dest_v)
        pltpu.sync_copy(x_hbm.at[pl.ds(base, _SUB)], rows_v.at[0])
        pltpu.sync_copy(sc16_hbm.at[pl.ds(base, _SUB)], srow_v.at[0])
        for si in range(_NSUB):
            cur = si % 2
            h1 = pltpu.async_copy(rows_v.at[cur], xs_hbm.at[dest_v.at[si]],
                                  sem)
            h2 = pltpu.async_copy(srow_v.at[cur], ss_hbm.at[dest_v.at[si]],
                                  sem)
            if si + 1 < _NSUB:
                nxt = (si + 1) % 2
                off = base + (si + 1) * _SUB
                pltpu.sync_copy(x_hbm.at[pl.ds(off, _SUB)], rows_v.at[nxt])
                pltpu.sync_copy(sc16_hbm.at[pl.ds(off, _SUB)], srow_v.at[nxt])
            h1.wait()
            h2.wait()

    @functools.partial(
        pl.kernel,
        mesh=mesh,
        out_type=jax.ShapeDtypeStruct((_N, _C), jnp.float32),
        scratch_types=[
            pltpu.VMEM((_NSUB, _SUB), jnp.int32),
            pltpu.VMEM((2, _SUB, _C), jnp.float32),
            pltpu.SemaphoreType.DMA,
            pltpu.SemaphoreType.DMA,
        ],
    )
    def combine(hs_hbm, dest_hbm, out_hbm, dest_v, rows_v, sem0, sem1):
        wid = lax.axis_index("s") * 2 + lax.axis_index("c")
        base = wid * _CHUNK
        sems = (sem0, sem1)
        pltpu.sync_copy(dest_hbm.at[wid], dest_v)
        pending = pltpu.async_copy(hs_hbm.at[dest_v.at[0]], rows_v.at[0],
                                   sems[0])
        for si in range(_NSUB):
            cur = si % 2
            if si + 1 < _NSUB:
                nxt_h = pltpu.async_copy(hs_hbm.at[dest_v.at[si + 1]],
                                         rows_v.at[(si + 1) % 2],
                                         sems[(si + 1) % 2])
            pending.wait()
            pltpu.sync_copy(rows_v.at[cur],
                            out_hbm.at[pl.ds(base + si * _SUB, _SUB)])
            if si + 1 < _NSUB:
                pending = nxt_h

    return dispatch, combine


_HH = _H // 2


def _expert_a_body(eid_ref, act_ref, xmap_ref, xs_ref, w1_ref, b1_ref, w2_ref,
                   out_ref):
    b = pl.program_id(0)

    @pl.when(act_ref[b] == 1)
    def _():
        h = lax.dot_general(xs_ref[...], w1_ref[0], _DN,
                            preferred_element_type=jnp.float32)
        h = jnp.maximum(h + b1_ref[0], 0.0)
        out_ref[...] = lax.dot_general(h, w2_ref[0], _DN,
                                       preferred_element_type=jnp.float32)


def _expert_b_body(eid_ref, act_ref, xmap_ref, xs_ref, ss_ref, prev_ref,
                   w1_ref, b1_ref, w2_ref, b2_ref, out_ref):
    b = pl.program_id(0)

    @pl.when(act_ref[b] == 1)
    def _():
        h = lax.dot_general(xs_ref[...], w1_ref[0], _DN,
                            preferred_element_type=jnp.float32)
        h = jnp.maximum(h + b1_ref[0], 0.0)
        o = lax.dot_general(h, w2_ref[0], _DN,
                            preferred_element_type=jnp.float32)
        out_ref[...] = (prev_ref[...] + o + b2_ref[0]) * ss_ref[:, 0:1]


_expert_a = pl.pallas_call(
    _expert_a_body,
    grid_spec=pltpu.PrefetchScalarGridSpec(
        num_scalar_prefetch=3,
        grid=(_NBLK,),
        in_specs=[
            pl.BlockSpec((_BLK, _C), lambda b, eid, act, xm: (xm[b], 0)),
            pl.BlockSpec((1, _C, _HH), lambda b, eid, act, xm: (eid[b], 0, 0)),
            pl.BlockSpec((1, 1, _HH), lambda b, eid, act, xm: (eid[b], 0, 0)),
            pl.BlockSpec((1, _HH, _C), lambda b, eid, act, xm: (eid[b], 0, 0)),
        ],
        out_specs=pl.BlockSpec((_BLK, _C), lambda b, eid, act, xm: (xm[b], 0)),
    ),
    out_shape=jax.ShapeDtypeStruct((_P, _C), jnp.float32),
)

_expert_b = pl.pallas_call(
    _expert_b_body,
    grid_spec=pltpu.PrefetchScalarGridSpec(
        num_scalar_prefetch=3,
        grid=(_NBLK,),
        in_specs=[
            pl.BlockSpec((_BLK, _C), lambda b, eid, act, xm: (xm[b], 0)),
            pl.BlockSpec((_BLK, 128), lambda b, eid, act, xm: (xm[b], 0)),
            pl.BlockSpec((_BLK, _C), lambda b, eid, act, xm: (xm[b], 0)),
            pl.BlockSpec((1, _C, _HH), lambda b, eid, act, xm: (eid[b], 0, 1)),
            pl.BlockSpec((1, 1, _HH), lambda b, eid, act, xm: (eid[b], 0, 1)),
            pl.BlockSpec((1, _HH, _C), lambda b, eid, act, xm: (eid[b], 1, 0)),
            pl.BlockSpec((1, 1, _C), lambda b, eid, act, xm: (eid[b], 0, 0)),
        ],
        out_specs=pl.BlockSpec((_BLK, _C), lambda b, eid, act, xm: (xm[b], 0)),
    ),
    out_shape=jax.ShapeDtypeStruct((_P, _C), jnp.float32),
)


def kernel(x, gate_W, gate_b, W1, b1, W2, b2):
    x2d = x.reshape(_N, _C)
    score_col, dest_col, cnt, aux = _gate(x2d, gate_W, gate_b.reshape(1, _E))
    counts = cnt.reshape(_E)
    nb = (counts + (_BLK - 1)) // _BLK
    cumnb = jnp.cumsum(nb)
    total = cumnb[-1].astype(jnp.int32)
    bids = jnp.arange(_NBLK, dtype=jnp.int32)
    eid_raw = jnp.sum((bids[:, None] >= cumnb[None, :]).astype(jnp.int32),
                      axis=1)
    eid_last = jnp.sum((cumnb <= total - 1).astype(jnp.int32))
    active = (bids < total).astype(jnp.int32)
    eid = jnp.where(active == 1, eid_raw, eid_last).astype(jnp.int32)
    xmap = jnp.where(active == 1, bids, total - 1).astype(jnp.int32)

    dest3 = dest_col.reshape(_NW, _NSUB, _SUB)
    _dispatch, _combine = _sc_kernels()
    xs, ss = _dispatch(x2d, score_col, dest3)
    b1r = b1.reshape(_E, 1, _H)
    hs_a = _expert_a(eid, active, xmap, xs, W1, b1r, W2)
    hs = _expert_b(eid, active, xmap, xs, ss, hs_a, W1, b1r, W2,
                   b2.reshape(_E, 1, _C))
    hout = _combine(hs, dest3)
    return (hout.reshape(_B, _T, _C), aux[0, 0])
